# baseline (speedup 1.0000x reference)
import functools

import jax
import jax.numpy as jnp
from jax import lax
from jax.experimental import pallas as pl
from jax.experimental.pallas import tpu as pltpu
from jax.experimental.pallas import tpu_sc as plsc

NUM_VARS = 26
VOCAB = 100000
D_MODEL = 32
NUM_PATCHES = 16
BATCH = 4096

NUM_CORES = 2
NUM_SUBCORES = 16
NUM_WORKERS = NUM_CORES * NUM_SUBCORES
BB = BATCH // NUM_WORKERS
LANES = 16


def _emb_body(idx_hbm, tab_hbm, out_hbm, idx_l, idx_g, rows, lsem, gsem, ssem):
    wid = lax.axis_index("s") * NUM_CORES + lax.axis_index("c")
    b0 = wid * BB

    pltpu.async_copy(idx_hbm.at[:, pl.ds(b0, BB)], idx_l, lsem).wait()

    for v in range(NUM_VARS):
        for k in range(BB // LANES):
            sl = pl.ds(k * LANES, LANES)
            idx_g[v, sl] = idx_l[v, sl] + v * VOCAB

    def gather(v, buf):
        return pltpu.async_copy(tab_hbm.at[idx_g.at[v]], rows.at[buf], gsem)

    g = gather(0, 0)
    pending = [[], []]
    for v in range(NUM_VARS):
        buf = v % 2
        g.wait()
        if v + 1 < NUM_VARS:
            nbuf = (v + 1) % 2
            for h in pending[nbuf]:
                h.wait()
            pending[nbuf] = []
            g = gather(v + 1, nbuf)
        hs = []
        for p in range(NUM_PATCHES):
            hs.append(
                pltpu.async_copy(
                    rows.at[buf], out_hbm.at[pl.ds(b0, BB), v, p], ssem
                )
            )
        pending[buf] = hs
    for hs in pending:
        for h in hs:
            h.wait()


@jax.jit
def _emb_call(idx_t, tab_flat):
    mesh = plsc.VectorSubcoreMesh(core_axis_name="c", subcore_axis_name="s")
    return pl.kernel(
        _emb_body,
        out_type=jax.ShapeDtypeStruct(
            (BATCH, NUM_VARS, NUM_PATCHES, D_MODEL), jnp.float32
        ),
        mesh=mesh,
        compiler_params=pltpu.CompilerParams(use_tc_tiling_on_sc=False),
        scratch_types=[
            pltpu.VMEM((NUM_VARS, BB), jnp.int32),
            pltpu.VMEM((NUM_VARS, BB), jnp.int32),
            pltpu.VMEM((2, BB, D_MODEL), jnp.float32),
            pltpu.SemaphoreType.DMA,
            pltpu.SemaphoreType.DMA,
            pltpu.SemaphoreType.DMA,
        ],
    )(idx_t, tab_flat)


def kernel(static_categorical_values, tables):
    idx_t = jnp.transpose(static_categorical_values.astype(jnp.int32))
    tab_flat = tables.reshape(NUM_VARS * VOCAB, D_MODEL)
    return _emb_call(idx_t, tab_flat)
